# baseline (speedup 1.0000x reference)
import functools

import jax
import jax.numpy as jnp
from jax import lax
from jax.experimental import pallas as pl
from jax.experimental.pallas import tpu as pltpu
from jax.experimental.pallas import tpu_sc as plsc

_LANES = 16
_NCHUNK = 1


@functools.lru_cache(maxsize=None)
def _make_sc_kernel(batch: int):
    info = plsc.get_sparse_core_info()
    nc, ns = info.num_cores, info.num_subcores
    nw = nc * ns
    assert batch % (8 * nw) == 0
    b_per_w = batch // nw
    chunk = b_per_w // _NCHUNK

    mesh = plsc.VectorSubcoreMesh(core_axis_name="c", subcore_axis_name="s")

    @functools.partial(
        pl.kernel,
        mesh=mesh,
        out_type=jax.ShapeDtypeStruct((batch,), jnp.float32),
        scratch_types=[
            pltpu.VMEM((b_per_w,), jnp.int32),
            pltpu.VMEM((b_per_w,), jnp.float32),
            pltpu.VMEM((_LANES,), jnp.float32),
            pltpu.SemaphoreType.DMA,
            pltpu.SemaphoreType.DMA,
        ]
        + [pltpu.SemaphoreType.DMA for _ in range(_NCHUNK)],
    )
    def sc_kernel(dls_hbm, w_hbm, b_hbm, idx_hbm, out_hbm,
                  idx_v, vals_v, wb_v, sem_in, sem_out, *sem_g):
        wid = lax.axis_index("s") * nc + lax.axis_index("c")
        base = wid * b_per_w
        cp_w = pltpu.async_copy(w_hbm.at[0], wb_v.at[pl.ds(0, 1)], sem_in)
        cp_b = pltpu.async_copy(b_hbm, wb_v.at[pl.ds(8, 1)], sem_in)
        cp_i = pltpu.async_copy(idx_hbm.at[pl.ds(base, b_per_w)], idx_v,
                                sem_in)
        cp_w.wait()
        cp_b.wait()
        cp_i.wait()
        gathers = []
        for k in range(_NCHUNK):
            sl = pl.ds(k * chunk, chunk)
            gathers.append(
                pltpu.async_copy(dls_hbm.at[idx_v.at[sl]], vals_v.at[sl],
                                 sem_g[k]))
        wb_vec = wb_v[...]
        w_vec = jnp.full((_LANES,), wb_vec[0], dtype=jnp.float32)
        b_vec = jnp.full((_LANES,), wb_vec[8], dtype=jnp.float32)
        stores = []
        for k in range(_NCHUNK):
            gathers[k].wait()
            for i in range(chunk // _LANES):
                sl = pl.ds(k * chunk + i * _LANES, _LANES)
                vals_v[sl] = vals_v[sl] * w_vec + b_vec
            sl = pl.ds(k * chunk, chunk)
            stores.append(
                pltpu.async_copy(vals_v.at[sl],
                                 out_hbm.at[pl.ds(base + k * chunk, chunk)],
                                 sem_out))
        for st in stores:
            st.wait()

    return sc_kernel


def kernel(dls, w, b, cells_oi):
    batch = cells_oi.shape[0]
    idx = cells_oi.astype(jnp.int32)
    out = _make_sc_kernel(batch)(dls, w, b, idx)
    return out.reshape(-1, 1)


# speedup vs baseline: 1.0148x; 1.0148x over previous
import functools

import jax
import jax.numpy as jnp
from jax import lax
from jax.experimental import pallas as pl
from jax.experimental.pallas import tpu as pltpu
from jax.experimental.pallas import tpu_sc as plsc

_LANES = 16
_NCHUNK = 2


@functools.lru_cache(maxsize=None)
def _make_sc_kernel(batch: int):
    info = plsc.get_sparse_core_info()
    nc, ns = info.num_cores, info.num_subcores
    nw = nc * ns
    assert batch % (8 * nw) == 0
    b_per_w = batch // nw
    chunk = b_per_w // _NCHUNK

    mesh = plsc.VectorSubcoreMesh(core_axis_name="c", subcore_axis_name="s")

    @functools.partial(
        pl.kernel,
        mesh=mesh,
        out_type=jax.ShapeDtypeStruct((batch,), jnp.float32),
        scratch_types=[
            pltpu.VMEM((b_per_w,), jnp.int32),
            pltpu.VMEM((b_per_w,), jnp.float32),
            pltpu.VMEM((_LANES,), jnp.float32),
            pltpu.SemaphoreType.DMA,
            pltpu.SemaphoreType.DMA,
        ]
        + [pltpu.SemaphoreType.DMA for _ in range(_NCHUNK)],
    )
    def sc_kernel(dls_hbm, w_hbm, b_hbm, idx_hbm, out_hbm,
                  idx_v, vals_v, wb_v, sem_in, sem_out, *sem_g):
        wid = lax.axis_index("s") * nc + lax.axis_index("c")
        base = wid * b_per_w
        cp_w = pltpu.async_copy(w_hbm.at[0], wb_v.at[pl.ds(0, 1)], sem_in)
        cp_b = pltpu.async_copy(b_hbm, wb_v.at[pl.ds(8, 1)], sem_in)
        cp_i = pltpu.async_copy(idx_hbm.at[pl.ds(base, b_per_w)], idx_v,
                                sem_in)
        cp_i.wait()
        gathers = []
        for k in range(_NCHUNK):
            sl = pl.ds(k * chunk, chunk)
            gathers.append(
                pltpu.async_copy(dls_hbm.at[idx_v.at[sl]], vals_v.at[sl],
                                 sem_g[k]))
        cp_w.wait()
        cp_b.wait()
        wb_vec = wb_v[...]
        w_vec = jnp.full((_LANES,), wb_vec[0], dtype=jnp.float32)
        b_vec = jnp.full((_LANES,), wb_vec[8], dtype=jnp.float32)
        stores = []
        for k in range(_NCHUNK):
            gathers[k].wait()
            for i in range(chunk // _LANES):
                sl = pl.ds(k * chunk + i * _LANES, _LANES)
                vals_v[sl] = vals_v[sl] * w_vec + b_vec
            sl = pl.ds(k * chunk, chunk)
            stores.append(
                pltpu.async_copy(vals_v.at[sl],
                                 out_hbm.at[pl.ds(base + k * chunk, chunk)],
                                 sem_out))
        for st in stores:
            st.wait()

    return sc_kernel


def kernel(dls, w, b, cells_oi):
    batch = cells_oi.shape[0]
    idx = cells_oi.astype(jnp.int32)
    out = _make_sc_kernel(batch)(dls, w, b, idx)
    return out.reshape(-1, 1)
